# baseline (speedup 1.0000x reference)
import functools

import jax
import jax.numpy as jnp
from jax import lax
from jax.experimental import pallas as pl
from jax.experimental.pallas import tpu as pltpu
from jax.experimental.pallas import tpu_sc as plsc

N = 100000
B = 10000
K = 33
DIN = 128
DOUT = 64
DE = 16
H = 4
ALPHA = 0.2
CTW = DIN + 2 * H + 8

BP = 10240
NSPLIT = 2
BPS = BP // NSPLIT
NC, NS = 2, 16
NW = NC * NS
CH = 264

SB = 128
NB = 2000


def _ct_body(node_ref, edge_ref, aep_ref, aen_ref, out_ref):
    aep = aep_ref[...]
    aen = aen_ref[...]
    Ae = jnp.stack([aep[h] for h in range(H)]
                   + [aen[h] for h in range(H)], axis=-1)
    ep = jnp.dot(edge_ref[...], Ae, preferred_element_type=jnp.float32)
    out_ref[...] = jnp.concatenate(
        [node_ref[...], ep, jnp.zeros((NB, 8), jnp.float32)], axis=1)


def _ct_build(node_table, edge_table, a_e_pos, a_e_neg):
    full = lambda shape: pl.BlockSpec(shape, lambda i: tuple(0 for _ in shape))
    return pl.pallas_call(
        _ct_body,
        grid=(N // NB,),
        in_specs=[
            pl.BlockSpec((NB, DIN), lambda i: (i, 0)),
            pl.BlockSpec((NB, DE), lambda i: (i, 0)),
            full((H, DE)), full((H, DE)),
        ],
        out_specs=pl.BlockSpec((NB, CTW), lambda i: (i, 0)),
        out_shape=jax.ShapeDtypeStruct((N, CTW), jnp.float32),
    )(node_table, edge_table, a_e_pos, a_e_neg)


def _sc_gather(ids_flat, ct, edge_table):
    rows = ids_flat.shape[0]
    bps = rows // K
    RPW = rows // NW
    NCH = RPW // CH
    SRPW = bps // NW
    assert RPW % CH == 0 and NCH % 2 == 0 and SRPW % 8 == 0
    mesh = plsc.VectorSubcoreMesh(core_axis_name="c", subcore_axis_name="s")

    @functools.partial(
        pl.kernel,
        mesh=mesh,
        compiler_params=pltpu.CompilerParams(use_tc_tiling_on_sc=False),
        out_type=(
            jax.ShapeDtypeStruct((rows, CTW), jnp.float32),
            jax.ShapeDtypeStruct((bps, DE), jnp.float32),
        ),
        scratch_types=[
            pltpu.VMEM((RPW,), jnp.int32),
            pltpu.VMEM((SRPW,), jnp.int32),
            pltpu.VMEM((CH, CTW), jnp.float32),
            pltpu.VMEM((CH, CTW), jnp.float32),
            pltpu.VMEM((SRPW, DE), jnp.float32),
            pltpu.SemaphoreType.DMA,
            pltpu.SemaphoreType.DMA,
            pltpu.SemaphoreType.DMA,
        ],
    )
    def gather_kernel(ids_hbm, ct_hbm, edge_hbm, x_out, e_out,
                      idx_v, sidx_v, x0, x1, ebuf, sx0, sx1, se):
        wid = lax.axis_index("s") * NC + lax.axis_index("c")
        base = wid * RPW
        pltpu.sync_copy(ids_hbm.at[pl.ds(base, RPW)], idx_v)
        soff = 16 * bps + wid * SRPW
        pltpu.sync_copy(ids_hbm.at[pl.ds(soff, SRPW)], sidx_v)
        pltpu.async_copy(edge_hbm.at[sidx_v], ebuf, se)

        xbuf = (x0, x1)
        sx = (sx0, sx1)

        def start(c, b):
            isl = idx_v.at[pl.ds(pl.multiple_of(c * CH, CH), CH)]
            pltpu.async_copy(ct_hbm.at[isl], xbuf[b], sx[b])

        def finish(c, b):
            pltpu.make_async_copy(ct_hbm.at[pl.ds(0, CH)], xbuf[b], sx[b]).wait()
            off = pl.multiple_of(base + c * CH, CH)
            pltpu.sync_copy(xbuf[b], x_out.at[pl.ds(off, CH)])

        start(0, 0)
        start(1, 1)

        def body(p, carry):
            c = p * 2
            finish(c, 0)

            @pl.when(p < NCH // 2 - 1)
            def _():
                start(c + 2, 0)

            finish(c + 1, 1)

            @pl.when(p < NCH // 2 - 1)
            def _():
                start(c + 3, 1)

            return carry

        lax.fori_loop(0, NCH // 2, body, 0)

        pltpu.make_async_copy(edge_hbm.at[pl.ds(0, SRPW)], ebuf, se).wait()
        pltpu.sync_copy(ebuf, e_out.at[pl.ds(wid * SRPW, SRPW)])

    return gather_kernel(ids_flat, ct, edge_table)


def _tc_body(x_ref, Wp_ref, asp_ref, adp_ref, Wn_ref, asn_ref, adn_ref,
             out_ref, us_ref, ud_ref):
    Wp = Wp_ref[...]
    Wn = Wn_ref[...]

    @pl.when(pl.program_id(0) == 0)
    def _():
        asp = asp_ref[...]
        adp = adp_ref[...]
        asn = asn_ref[...]
        adn = adn_ref[...]
        us_ref[...] = jnp.stack([jnp.dot(Wp[h], asp[h]) for h in range(H)]
                                + [jnp.dot(Wn[h], asn[h]) for h in range(H)],
                                axis=-1)
        ud_ref[...] = jnp.stack([jnp.dot(Wp[h], adp[h]) for h in range(H)]
                                + [jnp.dot(Wn[h], adn[h]) for h in range(H)],
                                axis=-1)

    Us = us_ref[...]
    Ud = ud_ref[...]

    x = x_ref

    dn0 = (((0,), (0,)), ((), ()))
    eyeb = (lax.broadcasted_iota(jnp.int32, (SB, SB), 0) ==
            lax.broadcasted_iota(jnp.int32, (SB, SB), 1)).astype(jnp.float32)
    xTf = [lax.dot_general(x[k], eyeb, dn0, preferred_element_type=jnp.float32)
           for k in range(K)]
    xT = [t[0:DIN] for t in xTf]
    eT = [t[DIN:DIN + 2 * H] for t in xTf]

    D = lax.dot_general(Ud, xT[16], dn0,
                        preferred_element_type=jnp.float32)
    Ls = []
    for k in range(K):
        Sk = lax.dot_general(Us, xT[k], dn0,
                             preferred_element_type=jnp.float32)
        Lk = Sk + eT[k] + D
        Ls.append(jnp.where(Lk >= 0, Lk, ALPHA * Lk))

    mp = Ls[0]
    for k in range(1, 17):
        mp = jnp.maximum(mp, Ls[k])
    mn = Ls[17]
    for k in range(18, K):
        mn = jnp.maximum(mn, Ls[k])

    wps = [jnp.exp(Ls[k] - mp) for k in range(17)]
    wns = [jnp.exp(Ls[17 + k] - mn) for k in range(16)]
    Zp = wps[0]
    for t in wps[1:]:
        Zp = Zp + t
    Zn = wns[0]
    for t in wns[1:]:
        Zn = Zn + t
    rp = 1.0 / Zp
    rn = 1.0 / Zn

    accp = [None] * H
    accn = [None] * H
    for h0 in (0, 2):
        a0 = jnp.zeros((DIN, SB), jnp.float32)
        a1 = jnp.zeros((DIN, SB), jnp.float32)
        for k in range(17):
            xk = xT[k]
            a0 = a0 + wps[k][h0:h0 + 1, :] * xk
            a1 = a1 + wps[k][h0 + 1:h0 + 2, :] * xk
        accp[h0] = a0 * rp[h0:h0 + 1, :]
        accp[h0 + 1] = a1 * rp[h0 + 1:h0 + 2, :]
        b0 = jnp.zeros((DIN, SB), jnp.float32)
        b1 = jnp.zeros((DIN, SB), jnp.float32)
        for k in range(16):
            xk = xT[17 + k]
            b0 = b0 + wns[k][H + h0:H + h0 + 1, :] * xk
            b1 = b1 + wns[k][H + h0 + 1:H + h0 + 2, :] * xk
        accn[h0] = b0 * rn[H + h0:H + h0 + 1, :]
        accn[h0 + 1] = b1 * rn[H + h0 + 1:H + h0 + 2, :]

    acc = lax.dot_general(accp[0], Wp[0], dn0, preferred_element_type=jnp.float32)
    for h in range(1, H):
        acc = acc + lax.dot_general(accp[h], Wp[h], dn0,
                                    preferred_element_type=jnp.float32)
    for h in range(H):
        acc = acc + lax.dot_general(accn[h], Wn[h], dn0,
                                    preferred_element_type=jnp.float32)
    out_ref[...] = jnp.maximum(acc * (1.0 / H), 0.0)


def _tc_dense(x3, W_pos, a_src_pos, a_dst_pos, W_neg, a_src_neg, a_dst_neg):
    bp = x3.shape[1]
    full = lambda shape: pl.BlockSpec(shape, lambda i: tuple(0 for _ in shape))
    return pl.pallas_call(
        _tc_body,
        grid=(bp // SB,),
        in_specs=[
            pl.BlockSpec((K, SB, CTW), lambda i: (0, i, 0)),
            full((H, DIN, DOUT)), full((H, DOUT)), full((H, DOUT)),
            full((H, DIN, DOUT)), full((H, DOUT)), full((H, DOUT)),
        ],
        out_specs=pl.BlockSpec((SB, DOUT), lambda i: (i, 0)),
        out_shape=jax.ShapeDtypeStruct((bp, DOUT), jnp.float32),
        scratch_shapes=[
            pltpu.VMEM((DIN, 2 * H), jnp.float32),
            pltpu.VMEM((DIN, 2 * H), jnp.float32),
        ],
    )(x3, W_pos, a_src_pos, a_dst_pos, W_neg, a_src_neg, a_dst_neg)


def kernel(nodes, neighs_pos, neighs_neg, node_table, edge_table,
           W_pos, a_src_pos, a_dst_pos, a_e_pos,
           W_neg, a_src_neg, a_dst_neg, a_e_neg):
    ct = _ct_build(node_table, edge_table, a_e_pos, a_e_neg)
    ids = jnp.concatenate([
        neighs_pos.T.astype(jnp.int32),
        nodes[None, :].astype(jnp.int32),
        neighs_neg.T.astype(jnp.int32),
    ], axis=0)
    ids = jnp.pad(ids, ((0, 0), (0, BP - B)))
    hs, ees = [], []
    for s in range(NSPLIT):
        ids_s = ids[:, s * BPS:(s + 1) * BPS]
        x_flat, e_seed = _sc_gather(ids_s.reshape(-1), ct, edge_table)
        x3 = x_flat.reshape(K, BPS, CTW)
        h_s = _tc_dense(x3, W_pos, a_src_pos, a_dst_pos,
                        W_neg, a_src_neg, a_dst_neg)
        hs.append(h_s)
        ees.append(e_seed)
    h_full = hs[0] if NSPLIT == 1 else jnp.concatenate(hs, axis=0)
    ee_full = ees[0] if NSPLIT == 1 else jnp.concatenate(ees, axis=0)
    return h_full[:B], ee_full[:B]


# speedup vs baseline: 1.3912x; 1.3912x over previous
import functools

import jax
import jax.numpy as jnp
from jax import lax
from jax.experimental import pallas as pl
from jax.experimental.pallas import tpu as pltpu
from jax.experimental.pallas import tpu_sc as plsc

N = 100000
B = 10000
K = 33
DIN = 128
DOUT = 64
DE = 16
H = 4
ALPHA = 0.2

BP = 10240
NSPLIT = 2
BPS = BP // NSPLIT
NC, NS = 2, 16
NW = NC * NS
CH = 264

SB = 128


def _sc_gather_one(ids_flat, table, width, tc_tiling):
    rows = ids_flat.shape[0]
    RPW = rows // NW
    NCH = RPW // CH
    assert RPW % CH == 0 and NCH % 2 == 0
    mesh = plsc.VectorSubcoreMesh(core_axis_name="c", subcore_axis_name="s")

    @functools.partial(
        pl.kernel,
        mesh=mesh,
        compiler_params=pltpu.CompilerParams(use_tc_tiling_on_sc=tc_tiling),
        out_type=jax.ShapeDtypeStruct((rows, width), jnp.float32),
        scratch_types=[
            pltpu.VMEM((RPW,), jnp.int32),
            pltpu.VMEM((CH, width), jnp.float32),
            pltpu.VMEM((CH, width), jnp.float32),
            pltpu.SemaphoreType.DMA,
            pltpu.SemaphoreType.DMA,
        ],
    )
    def gather_kernel(ids_hbm, table_hbm, x_out, idx_v, x0, x1, sx0, sx1):
        wid = lax.axis_index("s") * NC + lax.axis_index("c")
        base = wid * RPW
        pltpu.sync_copy(ids_hbm.at[pl.ds(base, RPW)], idx_v)

        xbuf = (x0, x1)
        sx = (sx0, sx1)

        def start(c, b):
            isl = idx_v.at[pl.ds(pl.multiple_of(c * CH, CH), CH)]
            pltpu.async_copy(table_hbm.at[isl], xbuf[b], sx[b])

        def finish(c, b):
            pltpu.make_async_copy(table_hbm.at[pl.ds(0, CH)], xbuf[b], sx[b]).wait()
            off = pl.multiple_of(base + c * CH, CH)
            pltpu.sync_copy(xbuf[b], x_out.at[pl.ds(off, CH)])

        start(0, 0)
        start(1, 1)

        def body(p, carry):
            c = p * 2
            finish(c, 0)

            @pl.when(p < NCH // 2 - 1)
            def _():
                start(c + 2, 0)

            finish(c + 1, 1)

            @pl.when(p < NCH // 2 - 1)
            def _():
                start(c + 3, 1)

            return carry

        lax.fori_loop(0, NCH // 2, body, 0)

    return gather_kernel(ids_flat, table)


def _tc_body(x_ref, e_ref, Wp_ref, asp_ref, adp_ref, aep_ref,
             Wn_ref, asn_ref, adn_ref, aen_ref, out_ref, ee_ref,
             us_ref, ud_ref, ae_ref):
    Wp = Wp_ref[...]
    Wn = Wn_ref[...]

    @pl.when(pl.program_id(0) == 0)
    def _():
        asp = asp_ref[...]
        adp = adp_ref[...]
        aep = aep_ref[...]
        asn = asn_ref[...]
        adn = adn_ref[...]
        aen = aen_ref[...]
        us_ref[...] = jnp.stack([jnp.dot(Wp[h], asp[h]) for h in range(H)]
                                + [jnp.dot(Wn[h], asn[h]) for h in range(H)],
                                axis=-1)
        ud_ref[...] = jnp.stack([jnp.dot(Wp[h], adp[h]) for h in range(H)]
                                + [jnp.dot(Wn[h], adn[h]) for h in range(H)],
                                axis=-1)
        ae_ref[...] = jnp.stack([aep[h] for h in range(H)]
                                + [aen[h] for h in range(H)], axis=-1)

    Us = us_ref[...]
    Ud = ud_ref[...]
    Ae = ae_ref[...]

    x = x_ref
    e = e_ref

    dn0 = (((0,), (0,)), ((), ()))
    dn1 = (((0,), (1,)), ((), ()))
    eyeb = (lax.broadcasted_iota(jnp.int32, (SB, SB), 0) ==
            lax.broadcasted_iota(jnp.int32, (SB, SB), 1)).astype(jnp.float32)
    xT = [lax.dot_general(x[k], eyeb, dn0, preferred_element_type=jnp.float32)
          for k in range(K)]

    D = lax.dot_general(Ud, xT[16], dn0,
                        preferred_element_type=jnp.float32)
    Ls = []
    for k in range(K):
        Sk = lax.dot_general(Us, xT[k], dn0,
                             preferred_element_type=jnp.float32)
        Ek = lax.dot_general(Ae, e[k], dn1,
                             preferred_element_type=jnp.float32)
        Lk = Sk + Ek + D
        Ls.append(jnp.where(Lk >= 0, Lk, ALPHA * Lk))

    mp = Ls[0]
    for k in range(1, 17):
        mp = jnp.maximum(mp, Ls[k])
    mn = Ls[17]
    for k in range(18, K):
        mn = jnp.maximum(mn, Ls[k])

    wps = [jnp.exp(Ls[k] - mp) for k in range(17)]
    wns = [jnp.exp(Ls[17 + k] - mn) for k in range(16)]
    Zp = wps[0]
    for t in wps[1:]:
        Zp = Zp + t
    Zn = wns[0]
    for t in wns[1:]:
        Zn = Zn + t
    rp = 1.0 / Zp
    rn = 1.0 / Zn

    accp = [None] * H
    accn = [None] * H
    for h0 in (0, 2):
        a0 = jnp.zeros((DIN, SB), jnp.float32)
        a1 = jnp.zeros((DIN, SB), jnp.float32)
        for k in range(17):
            xk = xT[k]
            a0 = a0 + wps[k][h0:h0 + 1, :] * xk
            a1 = a1 + wps[k][h0 + 1:h0 + 2, :] * xk
        accp[h0] = a0 * rp[h0:h0 + 1, :]
        accp[h0 + 1] = a1 * rp[h0 + 1:h0 + 2, :]
        b0 = jnp.zeros((DIN, SB), jnp.float32)
        b1 = jnp.zeros((DIN, SB), jnp.float32)
        for k in range(16):
            xk = xT[17 + k]
            b0 = b0 + wns[k][H + h0:H + h0 + 1, :] * xk
            b1 = b1 + wns[k][H + h0 + 1:H + h0 + 2, :] * xk
        accn[h0] = b0 * rn[H + h0:H + h0 + 1, :]
        accn[h0 + 1] = b1 * rn[H + h0 + 1:H + h0 + 2, :]

    acc = lax.dot_general(accp[0], Wp[0], dn0, preferred_element_type=jnp.float32)
    for h in range(1, H):
        acc = acc + lax.dot_general(accp[h], Wp[h], dn0,
                                    preferred_element_type=jnp.float32)
    for h in range(H):
        acc = acc + lax.dot_general(accn[h], Wn[h], dn0,
                                    preferred_element_type=jnp.float32)
    out_ref[...] = jnp.maximum(acc * (1.0 / H), 0.0)
    ee_ref[...] = e_ref[16]


def _tc_dense(x3, e3, W_pos, a_src_pos, a_dst_pos, a_e_pos,
              W_neg, a_src_neg, a_dst_neg, a_e_neg):
    bp = x3.shape[1]
    full = lambda shape: pl.BlockSpec(shape, lambda i: tuple(0 for _ in shape))
    return pl.pallas_call(
        _tc_body,
        grid=(bp // SB,),
        in_specs=[
            pl.BlockSpec((K, SB, DIN), lambda i: (0, i, 0)),
            pl.BlockSpec((K, SB, DE), lambda i: (0, i, 0)),
            full((H, DIN, DOUT)), full((H, DOUT)), full((H, DOUT)), full((H, DE)),
            full((H, DIN, DOUT)), full((H, DOUT)), full((H, DOUT)), full((H, DE)),
        ],
        out_specs=[
            pl.BlockSpec((SB, DOUT), lambda i: (i, 0)),
            pl.BlockSpec((SB, DE), lambda i: (i, 0)),
        ],
        out_shape=(
            jax.ShapeDtypeStruct((bp, DOUT), jnp.float32),
            jax.ShapeDtypeStruct((bp, DE), jnp.float32),
        ),
        scratch_shapes=[
            pltpu.VMEM((DIN, 2 * H), jnp.float32),
            pltpu.VMEM((DIN, 2 * H), jnp.float32),
            pltpu.VMEM((DE, 2 * H), jnp.float32),
        ],
    )(x3, e3, W_pos, a_src_pos, a_dst_pos, a_e_pos,
      W_neg, a_src_neg, a_dst_neg, a_e_neg)


def kernel(nodes, neighs_pos, neighs_neg, node_table, edge_table,
           W_pos, a_src_pos, a_dst_pos, a_e_pos,
           W_neg, a_src_neg, a_dst_neg, a_e_neg):
    ids = jnp.concatenate([
        neighs_pos.T.astype(jnp.int32),
        nodes[None, :].astype(jnp.int32),
        neighs_neg.T.astype(jnp.int32),
    ], axis=0)
    ids = jnp.pad(ids, ((0, 0), (0, BP - B)))
    hs, ees = [], []
    for s in range(NSPLIT):
        ids_s = ids[:, s * BPS:(s + 1) * BPS].reshape(-1)
        x_flat = _sc_gather_one(ids_s, node_table, DIN, True)
        e_flat = _sc_gather_one(ids_s, edge_table, DE, False)
        x3 = x_flat.reshape(K, BPS, DIN)
        e3 = e_flat.reshape(K, BPS, DE)
        h_s, ee_s = _tc_dense(x3, e3, W_pos, a_src_pos, a_dst_pos, a_e_pos,
                              W_neg, a_src_neg, a_dst_neg, a_e_neg)
        hs.append(h_s)
        ees.append(ee_s)
    h_full = hs[0] if NSPLIT == 1 else jnp.concatenate(hs, axis=0)
    ee_full = ees[0] if NSPLIT == 1 else jnp.concatenate(ees, axis=0)
    return h_full[:B], ee_full[:B]
